# Initial kernel scaffold; baseline (speedup 1.0000x reference)
import jax
import jax.numpy as jnp
from jax.experimental import pallas as pl


def kernel(x, edge_index, W1, b1, W2, b2):
    raise NotImplementedError("write your pallas kernel here")


# speedup vs baseline: 4.7400x; 4.7400x over previous
import jax
import jax.numpy as jnp
from jax import lax
from jax.experimental import pallas as pl
from jax.experimental.pallas import tpu as pltpu
from jax.experimental.pallas import tpu_sc as plsc

N_NODES = 10000
N_EDGES = 320000
NFEAT = 128
NHID = 128
NCLASS = 64

NC = 2
NS = 16

N_PAD = 10240
ROWS_PER_TILE = N_PAD // NS
BLK = 128
E_PAD = 323584
EDGES_PER_W = E_PAD // (NC * NS)
BLOCKS_PER_W = EDGES_PER_W // BLK


def _make_agg(d_feat: int, with_deg: bool):
    mesh = plsc.VectorSubcoreMesh(core_axis_name="c", subcore_axis_name="s")

    out_type = [jax.ShapeDtypeStruct((NC, N_NODES, d_feat), jnp.float32)]
    scratch = [
        pltpu.VMEM((BLK,), jnp.int32),
        pltpu.VMEM((BLK,), jnp.int32),
        pltpu.VMEM((BLK, d_feat), jnp.float32),
        pltpu.VMEM_SHARED((N_PAD, d_feat), jnp.float32),
    ]
    if with_deg:
        out_type.append(jax.ShapeDtypeStruct((NC, N_NODES, 16), jnp.float32))
        scratch += [
            pltpu.VMEM((BLK, 16), jnp.float32),
            pltpu.VMEM_SHARED((N_PAD, 16), jnp.float32),
        ]

    def body(feat_hbm, src_hbm, dst_hbm, zeros_hbm, *rest):
        if with_deg:
            (zeros16_hbm, ones_hbm, part_hbm, degp_hbm,
             src_v, dst_v, rows_v, acc_sh, ones_v, deg_sh) = rest
        else:
            part_hbm, src_v, dst_v, rows_v, acc_sh = rest
        c = lax.axis_index("c")
        s = lax.axis_index("s")
        w = c * NS + s

        pltpu.sync_copy(zeros_hbm,
                        acc_sh.at[pl.ds(s * ROWS_PER_TILE, ROWS_PER_TILE)])
        if with_deg:
            pltpu.sync_copy(ones_hbm, ones_v)
            pltpu.sync_copy(zeros16_hbm,
                            deg_sh.at[pl.ds(s * ROWS_PER_TILE, ROWS_PER_TILE)])
        plsc.subcore_barrier()

        base_w = w * EDGES_PER_W

        @pl.loop(0, BLOCKS_PER_W)
        def _(i):
            base = base_w + i * BLK
            pltpu.sync_copy(src_hbm.at[pl.ds(base, BLK)], src_v)
            pltpu.sync_copy(dst_hbm.at[pl.ds(base, BLK)], dst_v)
            pltpu.sync_copy(feat_hbm.at[src_v], rows_v)
            pltpu.sync_copy(rows_v, acc_sh.at[dst_v], add=True)
            if with_deg:
                pltpu.sync_copy(ones_v, deg_sh.at[dst_v], add=True)

        plsc.subcore_barrier()

        r0 = s * ROWS_PER_TILE
        tail = N_NODES - (NS - 1) * ROWS_PER_TILE

        @pl.when(s < NS - 1)
        def _():
            pltpu.sync_copy(acc_sh.at[pl.ds(r0, ROWS_PER_TILE)],
                            part_hbm.at[c, pl.ds(r0, ROWS_PER_TILE)])
            if with_deg:
                pltpu.sync_copy(deg_sh.at[pl.ds(r0, ROWS_PER_TILE)],
                                degp_hbm.at[c, pl.ds(r0, ROWS_PER_TILE)])

        @pl.when(s == NS - 1)
        def _():
            pltpu.sync_copy(acc_sh.at[pl.ds((NS - 1) * ROWS_PER_TILE, tail)],
                            part_hbm.at[c, pl.ds((NS - 1) * ROWS_PER_TILE, tail)])
            if with_deg:
                pltpu.sync_copy(deg_sh.at[pl.ds((NS - 1) * ROWS_PER_TILE, tail)],
                                degp_hbm.at[c, pl.ds((NS - 1) * ROWS_PER_TILE, tail)])

    return pl.kernel(body, out_type=tuple(out_type), mesh=mesh,
                     scratch_types=scratch,
                     compiler_params=pltpu.CompilerParams(
                         use_tc_tiling_on_sc=False))


_agg1 = _make_agg(NFEAT, with_deg=True)
_agg2 = _make_agg(NCLASS, with_deg=False)

ROW_BLK = 2000


def _layer1_body(p_ref, degp_ref, w1t_ref, b1_ref, w2t_ref, z_ref):
    s = p_ref[0] + p_ref[1]
    deg = degp_ref[0, :, 0:1] + degp_ref[1, :, 0:1]
    mean = s / (deg + 1e-6)
    h = jnp.maximum(
        jnp.dot(mean, w1t_ref[...], preferred_element_type=jnp.float32)
        + b1_ref[...], 0.0)
    z_ref[...] = jnp.dot(h, w2t_ref[...], preferred_element_type=jnp.float32)


def _layer2_body(q_ref, degp_ref, b2_ref, out_ref):
    s = q_ref[0] + q_ref[1]
    deg = degp_ref[0, :, 0:1] + degp_ref[1, :, 0:1]
    t = s / (deg + 1e-6) + b2_ref[...]
    m = jnp.max(t, axis=1, keepdims=True)
    ls = jnp.log(jnp.sum(jnp.exp(t - m), axis=1, keepdims=True)) + m
    out_ref[...] = t - ls


def _tc_layer1(p, degp, w1t, b1, w2t):
    return pl.pallas_call(
        _layer1_body,
        grid=(N_NODES // ROW_BLK,),
        in_specs=[
            pl.BlockSpec((NC, ROW_BLK, NFEAT), lambda i: (0, i, 0)),
            pl.BlockSpec((NC, ROW_BLK, 16), lambda i: (0, i, 0)),
            pl.BlockSpec((NFEAT, NHID), lambda i: (0, 0)),
            pl.BlockSpec((1, NHID), lambda i: (0, 0)),
            pl.BlockSpec((NHID, NCLASS), lambda i: (0, 0)),
        ],
        out_specs=pl.BlockSpec((ROW_BLK, NCLASS), lambda i: (i, 0)),
        out_shape=jax.ShapeDtypeStruct((N_NODES, NCLASS), jnp.float32),
    )(p, degp, w1t, b1, w2t)


def _tc_layer2(q, degp, b2):
    return pl.pallas_call(
        _layer2_body,
        grid=(N_NODES // ROW_BLK,),
        in_specs=[
            pl.BlockSpec((NC, ROW_BLK, NCLASS), lambda i: (0, i, 0)),
            pl.BlockSpec((NC, ROW_BLK, 16), lambda i: (0, i, 0)),
            pl.BlockSpec((1, NCLASS), lambda i: (0, 0)),
        ],
        out_specs=pl.BlockSpec((ROW_BLK, NCLASS), lambda i: (i, 0)),
        out_shape=jax.ShapeDtypeStruct((N_NODES, NCLASS), jnp.float32),
    )(q, degp, b2)


def kernel(x, edge_index, W1, b1, W2, b2):
    n_extra = E_PAD - N_EDGES
    src = jnp.concatenate(
        [edge_index[0].astype(jnp.int32), jnp.zeros((n_extra,), jnp.int32)])
    dst = jnp.concatenate(
        [edge_index[1].astype(jnp.int32),
         jnp.full((n_extra,), N_NODES, jnp.int32)])

    zeros128 = jnp.zeros((ROWS_PER_TILE, NFEAT), jnp.float32)
    zeros64 = jnp.zeros((ROWS_PER_TILE, NCLASS), jnp.float32)
    zeros16 = jnp.zeros((ROWS_PER_TILE, 16), jnp.float32)
    ones16 = jnp.ones((BLK, 16), jnp.float32)

    p, degp = _agg1(x, src, dst, zeros128, zeros16, ones16)
    z = _tc_layer1(p, degp, W1.T, b1.reshape(1, NHID), W2.T)
    (q,) = _agg2(z, src, dst, zeros64)
    return _tc_layer2(q, degp, b2.reshape(1, NCLASS))
